# baseline (speedup 1.0000x reference)
import jax
import jax.numpy as jnp
from jax.experimental import pallas as pl

NETS = 100
D = 64
BATCH = 1024
P = 2
Q = 10
GRID = NETS // (P * Q)


def _block_diag(w_stacked):
    zeros = jnp.zeros((D, D), dtype=w_stacked.dtype)
    rows = []
    for p in range(P):
        blk = w_stacked[p * D:(p + 1) * D, :]
        row = [blk if q == p else zeros for q in range(P)]
        rows.append(jnp.concatenate(row, axis=1))
    return jnp.concatenate(rows, axis=0)


def _mm(a, b):
    return jax.lax.dot_general(
        a, b, (((1,), (0,)), ((), ())),
        precision=jax.lax.Precision.DEFAULT,
        preferred_element_type=jnp.float32)


BCHUNK = 512


def _step(x_ref, w0_ref, w1_ref, w2_ref, out_ref):
    for q in range(Q):
        sl = pl.ds(q * P * D, P * D)
        b0 = _block_diag(w0_ref[sl, :])
        b1 = _block_diag(w1_ref[sl, :])
        b2 = _block_diag(w2_ref[sl, :])
        m = _mm(b2, _mm(b1, b0))
        out_ref[sl, :] = _mm(m.astype(jnp.bfloat16),
                             x_ref[sl, :].astype(jnp.bfloat16))


@jax.jit
def kernel(x, w0, w1, w2):
    w0m = w0.reshape(NETS * D, D)
    w1m = w1.reshape(NETS * D, D)
    w2m = w2.reshape(NETS * D, D)
    wspec = pl.BlockSpec((Q * P * D, D), lambda i, j: (i, 0))
    xspec = pl.BlockSpec((Q * P * D, BCHUNK), lambda i, j: (i, j))
    return pl.pallas_call(
        _step,
        grid=(GRID, BATCH // BCHUNK),
        in_specs=[xspec, wspec, wspec, wspec],
        out_specs=xspec,
        out_shape=jax.ShapeDtypeStruct((NETS * D, BATCH), jnp.float32),
    )(x, w0m, w1m, w2m)


# speedup vs baseline: 1.2415x; 1.2415x over previous
import jax
import jax.numpy as jnp
from jax.experimental import pallas as pl
from jax.experimental.pallas import tpu as pltpu

NETS = 100
D = 64
BATCH = 1024
P = 2
Q = 5
GRID = NETS // (P * Q)


def _block_diag(w_stacked):
    zeros = jnp.zeros((D, D), dtype=w_stacked.dtype)
    rows = []
    for p in range(P):
        blk = w_stacked[p * D:(p + 1) * D, :]
        row = [blk if q == p else zeros for q in range(P)]
        rows.append(jnp.concatenate(row, axis=1))
    return jnp.concatenate(rows, axis=0)


def _mm(a, b):
    return jax.lax.dot_general(
        a, b, (((1,), (0,)), ((), ())),
        precision=jax.lax.Precision.DEFAULT,
        preferred_element_type=jnp.float32)


BCHUNK = 1024


def _step(x_ref, w0_ref, w1_ref, w2_ref, out_ref):
    for q in range(Q):
        sl = pl.ds(q * P * D, P * D)
        b0 = _block_diag(w0_ref[sl, :])
        b1 = _block_diag(w1_ref[sl, :])
        b2 = _block_diag(w2_ref[sl, :])
        m = _mm(b2, _mm(b1, b0))
        out_ref[sl, :] = _mm(m.astype(jnp.bfloat16),
                             x_ref[sl, :].astype(jnp.bfloat16))


@jax.jit
def kernel(x, w0, w1, w2):
    w0m = w0.reshape(NETS * D, D)
    w1m = w1.reshape(NETS * D, D)
    w2m = w2.reshape(NETS * D, D)
    wspec = pl.BlockSpec((Q * P * D, D), lambda i, j: (i, 0))
    xspec = pl.BlockSpec((Q * P * D, BCHUNK), lambda i, j: (i, j))
    return pl.pallas_call(
        _step,
        grid=(GRID, BATCH // BCHUNK),
        in_specs=[xspec, wspec, wspec, wspec],
        out_specs=xspec,
        out_shape=jax.ShapeDtypeStruct((NETS * D, BATCH), jnp.float32),
        compiler_params=pltpu.CompilerParams(
            dimension_semantics=("parallel", "parallel")),
    )(x, w0m, w1m, w2m)


# speedup vs baseline: 1.3434x; 1.0821x over previous
import jax
import jax.numpy as jnp
from jax.experimental import pallas as pl
from jax.experimental.pallas import tpu as pltpu

NETS = 100
D = 64
BATCH = 1024
P = 4
Q = 5
GRID = NETS // (P * Q)


def _block_diag(w_stacked):
    zeros = jnp.zeros((D, D), dtype=w_stacked.dtype)
    rows = []
    for p in range(P):
        blk = w_stacked[p * D:(p + 1) * D, :]
        row = [blk if q == p else zeros for q in range(P)]
        rows.append(jnp.concatenate(row, axis=1))
    return jnp.concatenate(rows, axis=0)


def _mm(a, b):
    return jax.lax.dot_general(
        a, b, (((1,), (0,)), ((), ())),
        precision=jax.lax.Precision.DEFAULT,
        preferred_element_type=jnp.float32)


BCHUNK = 1024


def _step(x_ref, w0_ref, w1_ref, w2_ref, out_ref):
    for q in range(Q):
        sl = pl.ds(q * P * D, P * D)
        b0 = _block_diag(w0_ref[sl, :])
        b1 = _block_diag(w1_ref[sl, :])
        b2 = _block_diag(w2_ref[sl, :])
        m = _mm(b2, _mm(b1, b0))
        out_ref[sl, :] = _mm(m.astype(jnp.bfloat16),
                             x_ref[sl, :].astype(jnp.bfloat16))


@jax.jit
def kernel(x, w0, w1, w2):
    w0m = w0.reshape(NETS * D, D)
    w1m = w1.reshape(NETS * D, D)
    w2m = w2.reshape(NETS * D, D)
    wspec = pl.BlockSpec((Q * P * D, D), lambda i, j: (i, 0))
    xspec = pl.BlockSpec((Q * P * D, BCHUNK), lambda i, j: (i, j))
    return pl.pallas_call(
        _step,
        grid=(GRID, BATCH // BCHUNK),
        in_specs=[xspec, wspec, wspec, wspec],
        out_specs=xspec,
        out_shape=jax.ShapeDtypeStruct((NETS * D, BATCH), jnp.float32),
        compiler_params=pltpu.CompilerParams(
            dimension_semantics=("parallel", "parallel")),
    )(x, w0m, w1m, w2m)
